# baseline (speedup 1.0000x reference)
import jax
from jax.experimental import pallas as pl
from jax.experimental.pallas import tpu as pltpu

_N_CHUNKS = 8


def _copy_body(x_ref, o_ref, sem):
    copy = pltpu.make_async_copy(x_ref, o_ref, sem)
    copy.start()
    copy.wait()


def kernel(x):
    rows, cols = x.shape
    xt = x.T
    out = pl.pallas_call(
        _copy_body,
        in_specs=[pl.BlockSpec(memory_space=pl.ANY)],
        out_specs=pl.BlockSpec(memory_space=pl.ANY),
        out_shape=jax.ShapeDtypeStruct((cols, rows), x.dtype),
        scratch_shapes=[pltpu.SemaphoreType.DMA],
    )(xt)
    return out.T


# speedup vs baseline: 34.9955x; 34.9955x over previous
import jax
from jax.experimental import pallas as pl
from jax.experimental.pallas import tpu as pltpu

_ROW_CHUNKS = 8
_COL_CHUNKS = 2
_N_CHUNKS = _ROW_CHUNKS * _COL_CHUNKS


def _chunk_slices(shape):
    r_step = shape[0] // _ROW_CHUNKS
    c_step = shape[1] // _COL_CHUNKS
    return [
        (pl.ds(r * r_step, r_step), pl.ds(c * c_step, c_step))
        for r in range(_ROW_CHUNKS)
        for c in range(_COL_CHUNKS)
    ]


def _copy_body(x_ref, o_ref, buf, in_sems, out_sems):
    slices = _chunk_slices(x_ref.shape)
    for i, sl in enumerate(slices):
        pltpu.make_async_copy(x_ref.at[sl], buf.at[sl], in_sems.at[i]).start()
    for i, sl in enumerate(slices):
        pltpu.make_async_copy(x_ref.at[sl], buf.at[sl], in_sems.at[i]).wait()
        pltpu.make_async_copy(buf.at[sl], o_ref.at[sl], out_sems.at[i]).start()
    for i, sl in enumerate(slices):
        pltpu.make_async_copy(buf.at[sl], o_ref.at[sl], out_sems.at[i]).wait()


def kernel(x):
    rows, cols = x.shape
    xt = x.T
    out = pl.pallas_call(
        _copy_body,
        in_specs=[pl.BlockSpec(memory_space=pl.ANY)],
        out_specs=pl.BlockSpec(memory_space=pl.ANY),
        out_shape=jax.ShapeDtypeStruct((cols, rows), x.dtype),
        scratch_shapes=[
            pltpu.VMEM((cols, rows), x.dtype),
            pltpu.SemaphoreType.DMA((_N_CHUNKS,)),
            pltpu.SemaphoreType.DMA((_N_CHUNKS,)),
        ],
    )(xt)
    return out.T


# speedup vs baseline: 36.5376x; 1.0441x over previous
import jax
from jax.experimental import pallas as pl
from jax.experimental.pallas import tpu as pltpu

_ROW_CHUNKS = 4
_COL_CHUNKS = 1
_N_CHUNKS = _ROW_CHUNKS * _COL_CHUNKS


def _chunk_slices(shape):
    r_step = shape[0] // _ROW_CHUNKS
    c_step = shape[1] // _COL_CHUNKS
    return [
        (pl.ds(r * r_step, r_step), pl.ds(c * c_step, c_step))
        for r in range(_ROW_CHUNKS)
        for c in range(_COL_CHUNKS)
    ]


def _copy_body(x_ref, o_ref, buf, in_sems, out_sems):
    slices = _chunk_slices(x_ref.shape)
    for i, sl in enumerate(slices):
        pltpu.make_async_copy(x_ref.at[sl], buf.at[sl], in_sems.at[i]).start()
    for i, sl in enumerate(slices):
        pltpu.make_async_copy(x_ref.at[sl], buf.at[sl], in_sems.at[i]).wait()
        pltpu.make_async_copy(buf.at[sl], o_ref.at[sl], out_sems.at[i]).start()
    for i, sl in enumerate(slices):
        pltpu.make_async_copy(buf.at[sl], o_ref.at[sl], out_sems.at[i]).wait()


def kernel(x):
    rows, cols = x.shape
    xt = x.T
    out = pl.pallas_call(
        _copy_body,
        in_specs=[pl.BlockSpec(memory_space=pl.ANY)],
        out_specs=pl.BlockSpec(memory_space=pl.ANY),
        out_shape=jax.ShapeDtypeStruct((cols, rows), x.dtype),
        scratch_shapes=[
            pltpu.VMEM((cols, rows), x.dtype),
            pltpu.SemaphoreType.DMA((_N_CHUNKS,)),
            pltpu.SemaphoreType.DMA((_N_CHUNKS,)),
        ],
    )(xt)
    return out.T
